# baseline (speedup 1.0000x reference)
import functools

import jax
import jax.numpy as jnp
from jax.experimental import pallas as pl
from jax.experimental.pallas import tpu as pltpu


def _body(H, W, a_ref, b_ref,
          map_a_ref, kx_a_ref, ky_a_ref, z_a_ref,
          map_b_ref, kx_b_ref, ky_b_ref, z_b_ref):
    HW = H * W
    t = jax.lax.broadcasted_iota(jnp.int32, (1, HW), 1)
    xs = (t & (W - 1)).astype(jnp.float32)
    ys = (t >> (W.bit_length() - 1)).astype(jnp.float32)

    def one(x_ref, map_ref, kx_ref, ky_ref, z_ref):
        R = x_ref[...]
        mn = jnp.min(R, axis=1, keepdims=True)
        mx = jnp.max(R, axis=1, keepdims=True)
        inv = 1.0 / (mx - mn)
        m = (R - mn) * inv
        map_ref[...] = m
        z = jnp.sum(m, axis=1, keepdims=True)
        kx = jnp.sum(m * xs, axis=1, keepdims=True)
        ky = jnp.sum(m * ys, axis=1, keepdims=True)
        kx_ref[...] = jnp.round(kx / z)
        ky_ref[...] = jnp.round(ky / z)
        z_ref[...] = z

    one(a_ref, map_a_ref, kx_a_ref, ky_a_ref, z_a_ref)
    one(b_ref, map_b_ref, kx_b_ref, ky_b_ref, z_b_ref)


@jax.jit
def kernel(combined_hm_preds, tf_combined_hm_preds):
    B, C, H, W = combined_hm_preds.shape
    BC, HW = B * C, H * W
    CB = 32
    nb = BC // CB

    a2 = combined_hm_preds.reshape(BC, HW)
    b2 = tf_combined_hm_preds.reshape(BC, HW)

    big_spec = pl.BlockSpec((CB, HW), lambda g: (g, 0))
    small_spec = pl.BlockSpec((CB, 1), lambda g: (g, 0))
    big_shape = jax.ShapeDtypeStruct((BC, HW), jnp.float32)
    small_shape = jax.ShapeDtypeStruct((BC, 1), jnp.float32)

    outs = pl.pallas_call(
        functools.partial(_body, H, W),
        grid=(nb,),
        in_specs=[big_spec, big_spec],
        out_specs=(big_spec, small_spec, small_spec, small_spec,
                   big_spec, small_spec, small_spec, small_spec),
        out_shape=(big_shape, small_shape, small_shape, small_shape,
                   big_shape, small_shape, small_shape, small_shape),
        compiler_params=pltpu.CompilerParams(
            dimension_semantics=("parallel",),
        ),
        name="minmax_centroid_fused",
    )(a2, b2)

    map_a, kx_a, ky_a, z_a, map_b, kx_b, ky_b, _z_b = outs

    map_val_all = map_a.reshape(B, C, H, W)
    tf_map_val_all = map_b.reshape(B, C, H, W)
    keypoint = jnp.stack([kx_a.reshape(B, C), ky_a.reshape(B, C)], axis=2)
    tf_keypoint = jnp.stack([kx_b.reshape(B, C), ky_b.reshape(B, C)], axis=2)
    get_zeta = z_a.reshape(B, C)
    return (map_val_all, keypoint, get_zeta, tf_map_val_all, tf_keypoint)


# speedup vs baseline: 3.6435x; 3.6435x over previous
import functools

import jax
import jax.numpy as jnp
from jax.experimental import pallas as pl
from jax.experimental.pallas import tpu as pltpu


def _body(H, W, a_ref, b_ref,
          map_a_ref, kp_a_ref, z_a_ref,
          map_b_ref, kp_b_ref, z_b_ref):
    xs = jax.lax.broadcasted_iota(jnp.int32, (1, 1, W), 2).astype(jnp.float32)
    ys = jax.lax.broadcasted_iota(jnp.int32, (1, H, 1), 1).astype(jnp.float32)

    def one(x_ref, map_ref, kp_ref, z_ref):
        R = x_ref[0]
        mn = jnp.min(R, axis=(1, 2), keepdims=True)
        mx = jnp.max(R, axis=(1, 2), keepdims=True)
        inv = 1.0 / (mx - mn)
        m = (R - mn) * inv
        map_ref[0] = m
        z = jnp.sum(m, axis=(1, 2), keepdims=True)
        kx = jnp.sum(m * xs, axis=(1, 2), keepdims=True)
        ky = jnp.sum(m * ys, axis=(1, 2), keepdims=True)
        kp_ref[0] = jnp.concatenate(
            [jnp.round(kx[:, 0, :] / z[:, 0, :]),
             jnp.round(ky[:, 0, :] / z[:, 0, :])], axis=1)
        z_ref[0] = z[:, 0, :]

    one(a_ref, map_a_ref, kp_a_ref, z_a_ref)
    one(b_ref, map_b_ref, kp_b_ref, z_b_ref)


@jax.jit
def kernel(combined_hm_preds, tf_combined_hm_preds):
    B, C, H, W = combined_hm_preds.shape
    CB = 32
    nc = C // CB

    big_spec = pl.BlockSpec((1, CB, H, W), lambda b, c: (b, c, 0, 0))
    kp_spec = pl.BlockSpec((1, CB, 2), lambda b, c: (b, c, 0))
    z_spec = pl.BlockSpec((1, CB, 1), lambda b, c: (b, c, 0))
    big_shape = jax.ShapeDtypeStruct((B, C, H, W), jnp.float32)
    kp_shape = jax.ShapeDtypeStruct((B, C, 2), jnp.float32)
    z_shape = jax.ShapeDtypeStruct((B, C, 1), jnp.float32)

    outs = pl.pallas_call(
        functools.partial(_body, H, W),
        grid=(B, nc),
        in_specs=[big_spec, big_spec],
        out_specs=(big_spec, kp_spec, z_spec,
                   big_spec, kp_spec, z_spec),
        out_shape=(big_shape, kp_shape, z_shape,
                   big_shape, kp_shape, z_shape),
        compiler_params=pltpu.CompilerParams(
            dimension_semantics=("parallel", "parallel"),
        ),
        name="minmax_centroid_fused",
    )(combined_hm_preds, tf_combined_hm_preds)

    map_val_all, keypoint, z_a, tf_map_val_all, tf_keypoint, _z_b = outs
    get_zeta = z_a.reshape(B, C)
    return (map_val_all, keypoint, get_zeta, tf_map_val_all, tf_keypoint)


# speedup vs baseline: 3.8149x; 1.0470x over previous
import functools

import jax
import jax.numpy as jnp
from jax.experimental import pallas as pl
from jax.experimental.pallas import tpu as pltpu


def _body(H, W, a_ref, b_ref,
          map_a_ref, kp_a_ref, z_a_ref,
          map_b_ref, kp_b_ref, z_b_ref):
    xs = jax.lax.broadcasted_iota(jnp.int32, (1, 1, W), 2).astype(jnp.float32)
    ys = jax.lax.broadcasted_iota(jnp.int32, (1, H, 1), 1).astype(jnp.float32)

    def one(x_ref, map_ref, kp_ref, z_ref):
        R = x_ref[0]
        mn = jnp.min(R, axis=(1, 2), keepdims=True)
        mx = jnp.max(R, axis=(1, 2), keepdims=True)
        inv = 1.0 / (mx - mn)
        m = (R - mn) * inv
        map_ref[0] = m
        z = jnp.sum(m, axis=(1, 2), keepdims=True)
        kx = jnp.sum(m * xs, axis=(1, 2), keepdims=True)
        ky = jnp.sum(m * ys, axis=(1, 2), keepdims=True)
        kp_ref[0] = jnp.concatenate(
            [jnp.round(kx[:, 0, :] / z[:, 0, :]),
             jnp.round(ky[:, 0, :] / z[:, 0, :])], axis=1)
        z_ref[0] = z[:, 0, :]

    one(a_ref, map_a_ref, kp_a_ref, z_a_ref)
    one(b_ref, map_b_ref, kp_b_ref, z_b_ref)


@jax.jit
def kernel(combined_hm_preds, tf_combined_hm_preds):
    B, C, H, W = combined_hm_preds.shape
    CB = 64
    nc = C // CB

    big_spec = pl.BlockSpec((1, CB, H, W), lambda b, c: (b, c, 0, 0))
    kp_spec = pl.BlockSpec((1, CB, 2), lambda b, c: (b, c, 0))
    z_spec = pl.BlockSpec((1, CB, 1), lambda b, c: (b, c, 0))
    big_shape = jax.ShapeDtypeStruct((B, C, H, W), jnp.float32)
    kp_shape = jax.ShapeDtypeStruct((B, C, 2), jnp.float32)
    z_shape = jax.ShapeDtypeStruct((B, C, 1), jnp.float32)

    outs = pl.pallas_call(
        functools.partial(_body, H, W),
        grid=(B, nc),
        in_specs=[big_spec, big_spec],
        out_specs=(big_spec, kp_spec, z_spec,
                   big_spec, kp_spec, z_spec),
        out_shape=(big_shape, kp_shape, z_shape,
                   big_shape, kp_shape, z_shape),
        compiler_params=pltpu.CompilerParams(
            dimension_semantics=("parallel", "parallel"),
            vmem_limit_bytes=56 * 1024 * 1024,
        ),
        name="minmax_centroid_fused",
    )(combined_hm_preds, tf_combined_hm_preds)

    map_val_all, keypoint, z_a, tf_map_val_all, tf_keypoint, _z_b = outs
    get_zeta = z_a.reshape(B, C)
    return (map_val_all, keypoint, get_zeta, tf_map_val_all, tf_keypoint)
